# baseline (speedup 1.0000x reference)
import functools

import jax
import jax.numpy as jnp
from jax.experimental import pallas as pl


def _matmul_block(a_ref, f_ref, o_ref):
    a = a_ref[...].astype(jnp.bfloat16)
    f = f_ref[...].astype(jnp.bfloat16)
    o_ref[...] = jnp.dot(a, f, preferred_element_type=jnp.float32)


@functools.partial(jax.jit, static_argnames=())
def kernel(features, A):
    if features.ndim != 2:
        raise RuntimeError('the dimension of features should be 2')
    M, K = A.shape
    _, N = features.shape
    BM = 128
    return pl.pallas_call(
        _matmul_block,
        grid=(M // BM,),
        in_specs=[
            pl.BlockSpec((BM, K), lambda i: (i, 0)),
            pl.BlockSpec((K, N), lambda i: (0, 0)),
        ],
        out_specs=pl.BlockSpec((BM, N), lambda i: (i, 0)),
        out_shape=jax.ShapeDtypeStruct((M, N), jnp.float32),
    )(A, features)


# speedup vs baseline: 1.2298x; 1.2298x over previous
import functools

import jax
import jax.numpy as jnp
from jax.experimental import pallas as pl
from jax.experimental.pallas import tpu as pltpu


def _matmul_block(a_ref, f_ref, o_ref):
    a = a_ref[...].astype(jnp.bfloat16)
    f = f_ref[...].astype(jnp.bfloat16)
    o_ref[...] = jnp.dot(a, f, preferred_element_type=jnp.float32)


@functools.partial(jax.jit, static_argnames=())
def kernel(features, A):
    if features.ndim != 2:
        raise RuntimeError('the dimension of features should be 2')
    M, K = A.shape
    _, N = features.shape
    BM = 256
    return pl.pallas_call(
        _matmul_block,
        grid=(M // BM,),
        in_specs=[
            pl.BlockSpec((BM, K), lambda i: (i, 0)),
            pl.BlockSpec((K, N), lambda i: (0, 0)),
        ],
        out_specs=pl.BlockSpec((BM, N), lambda i: (i, 0)),
        out_shape=jax.ShapeDtypeStruct((M, N), jnp.float32),
        compiler_params=pltpu.CompilerParams(
            dimension_semantics=("parallel",),
        ),
    )(A, features)
